# baseline (speedup 1.0000x reference)
import jax
import jax.numpy as jnp
from jax.experimental import pallas as pl
from jax.experimental.pallas import tpu as pltpu

_P, _N, _NFEAT, _NHID, _SHID = 3, 4096, 128, 64, 32
_BM = 1024
_NM = _N // _BM


def _fused_body(adj0_ref, adj1_ref, adj2_ref, adj3_ref,
                s1_ref, s2_ref, wg_ref, b_ref, a_ref,
                msk_ref, mskm_ref, sb1_ref, sb2_ref, l1_ref, l2_ref,
                wa_ref, ba_ref, qa_ref, wdt_ref, bd_ref,
                hh1_ref, loss_ref,
                f1_s, f2_s, h1_s, h2_s, t1_s, t2_s, r1_s):
    i = pl.program_id(0)
    m = pl.program_id(1)

    @pl.when(m == 0)
    def _init():
        wj = wg_ref[0]
        f1_s[...] = jnp.dot(
            s1_ref[0], wj,
            preferred_element_type=jnp.float32).astype(jnp.bfloat16)
        f2_s[...] = jnp.dot(
            s2_ref[0], wj,
            preferred_element_type=jnp.float32).astype(jnp.bfloat16)

    @pl.when((i == 0) & (m == 0))
    def _zero():
        t1_s[...] = jnp.zeros_like(t1_s)
        t2_s[...] = jnp.zeros_like(t2_s)
        r1_s[...] = jnp.zeros_like(r1_s)

    b = b_ref[0]
    a = a_ref[0]
    wa = wa_ref[...]
    ba = ba_ref[...]
    nq = _N // 4
    y1 = b.astype(jnp.float32)
    y2 = b.astype(jnp.float32)
    for q, aref in enumerate((adj0_ref, adj1_ref, adj2_ref, adj3_ref)):
        adj = aref[0, 0]
        fq1 = f1_s[q * nq:(q + 1) * nq, :].astype(jnp.float32)
        fq2 = f2_s[q * nq:(q + 1) * nq, :].astype(jnp.float32)
        y1 = y1 + jnp.dot(adj, fq1, preferred_element_type=jnp.float32)
        y2 = y2 + jnp.dot(adj, fq2, preferred_element_type=jnp.float32)
    h1 = jnp.where(y1 >= 0, y1, a * y1)
    h2 = jnp.where(y2 >= 0, y2, a * y2)
    hh1_ref[0] = h1
    h1_s[i, pl.ds(m * _BM, _BM), :] = h1.astype(jnp.bfloat16)
    h2_s[i, pl.ds(m * _BM, _BM), :] = h2.astype(jnp.bfloat16)
    u1 = jnp.tanh(jnp.dot(h1, wa, preferred_element_type=jnp.float32) + ba)
    u2 = jnp.tanh(jnp.dot(h2, wa, preferred_element_type=jnp.float32) + ba)
    t1_s[i] += jnp.sum(u1, axis=0, keepdims=True)
    t2_s[i] += jnp.sum(u2, axis=0, keepdims=True)
    r1_s[i] += jnp.dot(mskm_ref[0], h1, preferred_element_type=jnp.float32)

    @pl.when((i == _P - 1) & (m == _NM - 1))
    def _tail():
        qa = qa_ref[...]
        w1s = [jnp.sum(t1_s[j] * qa) / _N for j in range(_P)]
        w2s = [jnp.sum(t2_s[j] * qa) / _N for j in range(_P)]

        def _softmax3(ws):
            mx = jnp.maximum(jnp.maximum(ws[0], ws[1]), ws[2])
            es = [jnp.exp(w - mx) for w in ws]
            s = es[0] + es[1] + es[2]
            return [e / s for e in es]

        b1 = _softmax3(w1s)
        b2 = _softmax3(w2s)

        def _bce(x, t):
            return jnp.maximum(x, 0.0) - x * t + jnp.log1p(jnp.exp(-jnp.abs(x)))

        bd = bd_ref[0, 0]
        msk = msk_ref[...]

        r = b1[0] * r1_s[0] + b1[1] * r1_s[1] + b1[2] * r1_s[2]
        c = jax.nn.sigmoid(r / jnp.sum(msk))
        u = jnp.dot(c, wdt_ref[...], preferred_element_type=jnp.float32)
        ut = jnp.swapaxes(u, 0, 1)
        uhi = ut.astype(jnp.bfloat16)
        ulo = (ut - uhi.astype(jnp.float32)).astype(jnp.bfloat16)
        u2c = jnp.concatenate([uhi, ulo], axis=1)

        p1 = (b1[0] * jnp.dot(h1_s[0], u2c, preferred_element_type=jnp.float32)
              + b1[1] * jnp.dot(h1_s[1], u2c, preferred_element_type=jnp.float32)
              + b1[2] * jnp.dot(h1_s[2], u2c, preferred_element_type=jnp.float32))
        p2 = (b2[0] * jnp.dot(h2_s[0], u2c, preferred_element_type=jnp.float32)
              + b2[1] * jnp.dot(h2_s[1], u2c, preferred_element_type=jnp.float32)
              + b2[2] * jnp.dot(h2_s[2], u2c, preferred_element_type=jnp.float32))
        sc1 = p1[:, 0:1] + p1[:, 1:2]
        sc2 = p2[:, 0:1] + p2[:, 1:2]
        sc1 = sc1.reshape(_N // 128, 128) + bd + sb1_ref[...]
        sc2 = sc2.reshape(_N // 128, 128) + bd + sb2_ref[...]
        loss = (jnp.sum(_bce(sc1, l1_ref[...]), keepdims=True)
                + jnp.sum(_bce(sc2, l2_ref[...]), keepdims=True))
        loss_ref[...] = loss / (2 * _N)


def kernel(seq1, seq2, lbl, adjs, sparse, msk, samp_bias1, samp_bias2,
           W_gcn, b_gcn, a_prelu, W_att, b_att, q_att, W_disc, b_disc):
    del sparse
    b3 = b_gcn.reshape(_P, 1, _NHID)
    a3 = jnp.broadcast_to(a_prelu[:, None, None], (_P, 1, _NHID))
    const = lambda i, m: (0, 0)
    const3 = lambda i, m: (0, 0, 0)
    per_i = lambda i, m: (i, 0, 0)
    hh1, loss = pl.pallas_call(
        _fused_body,
        grid=(_P, _NM),
        in_specs=[
            pl.BlockSpec((1, 1, _BM, _N // 4), lambda i, m: (i, 0, m, 0)),
            pl.BlockSpec((1, 1, _BM, _N // 4), lambda i, m: (i, 0, m, 1)),
            pl.BlockSpec((1, 1, _BM, _N // 4), lambda i, m: (i, 0, m, 2)),
            pl.BlockSpec((1, 1, _BM, _N // 4), lambda i, m: (i, 0, m, 3)),
            pl.BlockSpec((1, _N, _NFEAT), const3),
            pl.BlockSpec((1, _N, _NFEAT), const3),
            pl.BlockSpec((1, _NFEAT, _NHID), per_i),
            pl.BlockSpec((1, 1, _NHID), per_i),
            pl.BlockSpec((1, 1, _NHID), per_i),
            pl.BlockSpec((1, _N), const),
            pl.BlockSpec((1, 1, _BM), lambda i, m: (m, 0, 0)),
            pl.BlockSpec((_N // 128, 128), const),
            pl.BlockSpec((_N // 128, 128), const),
            pl.BlockSpec((_N // 128, 128), const),
            pl.BlockSpec((_N // 128, 128), const),
            pl.BlockSpec((_NHID, _SHID), const),
            pl.BlockSpec((1, _SHID), const),
            pl.BlockSpec((1, _SHID), const),
            pl.BlockSpec((_NHID, _NHID), const),
            pl.BlockSpec((1, 1), const),
        ],
        out_specs=[
            pl.BlockSpec((1, _BM, _NHID), lambda i, m: (i, m, 0)),
            pl.BlockSpec((1, 1), const),
        ],
        out_shape=[
            jax.ShapeDtypeStruct((_P, _N, _NHID), jnp.float32),
            jax.ShapeDtypeStruct((1, 1), jnp.float32),
        ],
        scratch_shapes=[
            pltpu.VMEM((_N, _NHID), jnp.bfloat16),
            pltpu.VMEM((_N, _NHID), jnp.bfloat16),
            pltpu.VMEM((_P, _N, _NHID), jnp.bfloat16),
            pltpu.VMEM((_P, _N, _NHID), jnp.bfloat16),
            pltpu.VMEM((_P, 1, _SHID), jnp.float32),
            pltpu.VMEM((_P, 1, _SHID), jnp.float32),
            pltpu.VMEM((_P, 1, _NHID), jnp.float32),
        ],
    )(adjs, adjs, adjs, adjs, seq1, seq2, W_gcn, b3, a3,
      msk, msk.reshape(_NM, 1, _BM),
      samp_bias1.reshape(_N // 128, 128), samp_bias2.reshape(_N // 128, 128),
      lbl[:, :_N].reshape(_N // 128, 128), lbl[:, _N:].reshape(_N // 128, 128),
      W_att, b_att.reshape(1, _SHID), q_att.reshape(1, _SHID),
      W_disc.T, b_disc.reshape(1, 1))

    return (loss[0, 0], hh1)


# speedup vs baseline: 1.0115x; 1.0115x over previous
import jax
import jax.numpy as jnp
from jax.experimental import pallas as pl
from jax.experimental.pallas import tpu as pltpu

_P, _N, _NFEAT, _NHID, _SHID = 3, 4096, 128, 64, 32
_BM = 1024
_NM = _N // _BM


def _fused_body(adj0_ref, adj1_ref, adj2_ref, adj3_ref,
                s1_ref, s2_ref, wg_ref, b_ref, a_ref,
                msk_ref, sb1_ref, sb2_ref, l1_ref, l2_ref,
                wa_ref, ba_ref, qa_ref, wdt_ref, bd_ref,
                hh1_ref, loss_ref,
                f1_s, f2_s, h1_s, h2_s, t1_s, t2_s):
    i = pl.program_id(0)
    m = pl.program_id(1)

    @pl.when(m == 0)
    def _init():
        wj = wg_ref[0]
        f1_s[...] = jnp.dot(
            s1_ref[0], wj,
            preferred_element_type=jnp.float32).astype(jnp.bfloat16)
        f2_s[...] = jnp.dot(
            s2_ref[0], wj,
            preferred_element_type=jnp.float32).astype(jnp.bfloat16)

    @pl.when((i == 0) & (m == 0))
    def _zero():
        t1_s[...] = jnp.zeros_like(t1_s)
        t2_s[...] = jnp.zeros_like(t2_s)

    b = b_ref[0]
    a = a_ref[0]
    wa = wa_ref[...]
    ba = ba_ref[...]
    nq = _N // 4
    y1 = b.astype(jnp.float32)
    y2 = b.astype(jnp.float32)
    for q, aref in enumerate((adj0_ref, adj1_ref, adj2_ref, adj3_ref)):
        adj = aref[0, 0]
        fq1 = f1_s[q * nq:(q + 1) * nq, :].astype(jnp.float32)
        fq2 = f2_s[q * nq:(q + 1) * nq, :].astype(jnp.float32)
        y1 = y1 + jnp.dot(adj, fq1, preferred_element_type=jnp.float32)
        y2 = y2 + jnp.dot(adj, fq2, preferred_element_type=jnp.float32)
    h1 = jnp.where(y1 >= 0, y1, a * y1)
    h2 = jnp.where(y2 >= 0, y2, a * y2)
    hh1_ref[0] = h1
    h1_s[i, pl.ds(m * _BM, _BM), :] = h1.astype(jnp.bfloat16)
    h2_s[i, pl.ds(m * _BM, _BM), :] = h2.astype(jnp.bfloat16)
    u1 = jnp.tanh(jnp.dot(h1, wa, preferred_element_type=jnp.float32) + ba)
    u2 = jnp.tanh(jnp.dot(h2, wa, preferred_element_type=jnp.float32) + ba)
    t1_s[i] += jnp.sum(u1, axis=0, keepdims=True)
    t2_s[i] += jnp.sum(u2, axis=0, keepdims=True)

    @pl.when((i == _P - 1) & (m == _NM - 1))
    def _tail():
        qa = qa_ref[...]
        w1s = [jnp.sum(t1_s[j] * qa) / _N for j in range(_P)]
        w2s = [jnp.sum(t2_s[j] * qa) / _N for j in range(_P)]

        def _softmax3(ws):
            mx = jnp.maximum(jnp.maximum(ws[0], ws[1]), ws[2])
            es = [jnp.exp(w - mx) for w in ws]
            s = es[0] + es[1] + es[2]
            return [e / s for e in es]

        b1 = _softmax3(w1s)
        b2 = _softmax3(w2s)

        def _bce(x, t):
            return jnp.maximum(x, 0.0) - x * t + jnp.log1p(jnp.exp(-jnp.abs(x)))

        bd = bd_ref[0, 0]
        msk = msk_ref[...]
        mskb = msk.astype(jnp.bfloat16)

        r = (b1[0] * jnp.dot(mskb, h1_s[0], preferred_element_type=jnp.float32)
             + b1[1] * jnp.dot(mskb, h1_s[1], preferred_element_type=jnp.float32)
             + b1[2] * jnp.dot(mskb, h1_s[2], preferred_element_type=jnp.float32))
        c = jax.nn.sigmoid(r / jnp.sum(msk))
        u = jnp.dot(c, wdt_ref[...], preferred_element_type=jnp.float32)
        ut = jnp.swapaxes(u, 0, 1)
        uhi = ut.astype(jnp.bfloat16)
        ulo = (ut - uhi.astype(jnp.float32)).astype(jnp.bfloat16)
        u2c = jnp.concatenate([uhi, ulo], axis=1)

        p1 = (b1[0] * jnp.dot(h1_s[0], u2c, preferred_element_type=jnp.float32)
              + b1[1] * jnp.dot(h1_s[1], u2c, preferred_element_type=jnp.float32)
              + b1[2] * jnp.dot(h1_s[2], u2c, preferred_element_type=jnp.float32))
        p2 = (b2[0] * jnp.dot(h2_s[0], u2c, preferred_element_type=jnp.float32)
              + b2[1] * jnp.dot(h2_s[1], u2c, preferred_element_type=jnp.float32)
              + b2[2] * jnp.dot(h2_s[2], u2c, preferred_element_type=jnp.float32))
        sc1 = p1[:, 0:1] + p1[:, 1:2]
        sc2 = p2[:, 0:1] + p2[:, 1:2]
        sc1 = sc1.reshape(_N // 128, 128) + bd + sb1_ref[...]
        sc2 = sc2.reshape(_N // 128, 128) + bd + sb2_ref[...]
        loss = (jnp.sum(_bce(sc1, l1_ref[...]), keepdims=True)
                + jnp.sum(_bce(sc2, l2_ref[...]), keepdims=True))
        loss_ref[...] = loss / (2 * _N)


def kernel(seq1, seq2, lbl, adjs, sparse, msk, samp_bias1, samp_bias2,
           W_gcn, b_gcn, a_prelu, W_att, b_att, q_att, W_disc, b_disc):
    del sparse
    b3 = b_gcn.reshape(_P, 1, _NHID)
    a3 = jnp.broadcast_to(a_prelu[:, None, None], (_P, 1, _NHID))
    const = lambda i, m: (0, 0)
    const3 = lambda i, m: (0, 0, 0)
    per_i = lambda i, m: (i, 0, 0)
    hh1, loss = pl.pallas_call(
        _fused_body,
        grid=(_P, _NM),
        in_specs=[
            pl.BlockSpec((1, 1, _BM, _N // 4), lambda i, m: (i, 0, m, 0)),
            pl.BlockSpec((1, 1, _BM, _N // 4), lambda i, m: (i, 0, m, 1)),
            pl.BlockSpec((1, 1, _BM, _N // 4), lambda i, m: (i, 0, m, 2)),
            pl.BlockSpec((1, 1, _BM, _N // 4), lambda i, m: (i, 0, m, 3)),
            pl.BlockSpec((1, _N, _NFEAT), const3),
            pl.BlockSpec((1, _N, _NFEAT), const3),
            pl.BlockSpec((1, _NFEAT, _NHID), per_i),
            pl.BlockSpec((1, 1, _NHID), per_i),
            pl.BlockSpec((1, 1, _NHID), per_i),
            pl.BlockSpec((1, _N), const),
            pl.BlockSpec((_N // 128, 128), const),
            pl.BlockSpec((_N // 128, 128), const),
            pl.BlockSpec((_N // 128, 128), const),
            pl.BlockSpec((_N // 128, 128), const),
            pl.BlockSpec((_NHID, _SHID), const),
            pl.BlockSpec((1, _SHID), const),
            pl.BlockSpec((1, _SHID), const),
            pl.BlockSpec((_NHID, _NHID), const),
            pl.BlockSpec((1, 1), const),
        ],
        out_specs=[
            pl.BlockSpec((1, _BM, _NHID), lambda i, m: (i, m, 0)),
            pl.BlockSpec((1, 1), const),
        ],
        out_shape=[
            jax.ShapeDtypeStruct((_P, _N, _NHID), jnp.float32),
            jax.ShapeDtypeStruct((1, 1), jnp.float32),
        ],
        scratch_shapes=[
            pltpu.VMEM((_N, _NHID), jnp.bfloat16),
            pltpu.VMEM((_N, _NHID), jnp.bfloat16),
            pltpu.VMEM((_P, _N, _NHID), jnp.bfloat16),
            pltpu.VMEM((_P, _N, _NHID), jnp.bfloat16),
            pltpu.VMEM((_P, 1, _SHID), jnp.float32),
            pltpu.VMEM((_P, 1, _SHID), jnp.float32),
        ],
    )(adjs, adjs, adjs, adjs, seq1, seq2, W_gcn, b3, a3,
      msk,
      samp_bias1.reshape(_N // 128, 128), samp_bias2.reshape(_N // 128, 128),
      lbl[:, :_N].reshape(_N // 128, 128), lbl[:, _N:].reshape(_N // 128, 128),
      W_att, b_att.reshape(1, _SHID), q_att.reshape(1, _SHID),
      W_disc.T, b_disc.reshape(1, 1))

    return (loss[0, 0], hh1)
